# baseline (speedup 1.0000x reference)
import functools

import jax
import jax.numpy as jnp
from jax.experimental import pallas as pl
from jax.experimental.pallas import tpu as pltpu

K_NN = 3
BLOCK_N = 1024
N_CORES = 2


def _insert(v, ix, r0, r1, r2, j0, j1, j2):
    b0 = v < r0
    b1 = v < r1
    b2 = v < r2
    return (
        jnp.where(b0, v, r0),
        jnp.where(b0, r0, jnp.where(b1, v, r1)),
        jnp.where(b1, r1, jnp.where(b2, v, r2)),
        jnp.where(b0, ix, j0),
        jnp.where(b0, j0, jnp.where(b1, ix, j1)),
        jnp.where(b1, j1, jnp.where(b2, ix, j2)),
    )


def _knn_step(q_ref, m_ref, vals_ref, idx_ref, rv_ref, ri_ref, *,
              block_n, n_total, inner):
    core = pl.program_id(0)
    j = pl.program_id(1)

    @pl.when(j == 0)
    def _init():
        rv_ref[...] = jnp.full(rv_ref.shape, jnp.inf, jnp.float32)
        ri_ref[...] = jnp.zeros(ri_ref.shape, jnp.int32)

    q = q_ref[...]
    m = m_ref[...]
    qsq = jnp.sum(q * q, axis=1)
    msq = jnp.sum(m * m, axis=1)
    ab = jax.lax.dot_general(q, m, (((1,), (1,)), ((), ())),
                             preferred_element_type=jnp.float32)
    d2 = (qsq[:, None] + msq[None, :]) - 2.0 * ab
    dist = jnp.sqrt(jnp.maximum(d2, 1e-12))

    iota = jax.lax.broadcasted_iota(jnp.int32, dist.shape, 1)
    base = (core * inner + j) * block_n

    r0 = rv_ref[0, :]
    r1 = rv_ref[1, :]
    r2 = rv_ref[2, :]
    j0 = ri_ref[0, :]
    j1 = ri_ref[1, :]
    j2 = ri_ref[2, :]

    work = dist
    for _ in range(K_NN):
        mval = jnp.min(work, axis=1)
        hit = work == mval[:, None]
        midx = jnp.min(jnp.where(hit, iota, n_total), axis=1)
        work = jnp.where(iota == midx[:, None], jnp.inf, work)
        r0, r1, r2, j0, j1, j2 = _insert(mval, midx + base,
                                         r0, r1, r2, j0, j1, j2)

    rv_ref[0, :] = r0
    rv_ref[1, :] = r1
    rv_ref[2, :] = r2
    ri_ref[0, :] = j0
    ri_ref[1, :] = j1
    ri_ref[2, :] = j2

    @pl.when(j == inner - 1)
    def _finish():
        vals_ref[0, 0, :] = r0
        vals_ref[0, 1, :] = r1
        vals_ref[0, 2, :] = r2
        idx_ref[0, 0, :] = j0
        idx_ref[0, 1, :] = j1
        idx_ref[0, 2, :] = j2


def _merge_step(pv_ref, pi_ref, vals_ref, idx_ref):
    r0 = pv_ref[0, 0, :]
    r1 = pv_ref[0, 1, :]
    r2 = pv_ref[0, 2, :]
    j0 = pi_ref[0, 0, :]
    j1 = pi_ref[0, 1, :]
    j2 = pi_ref[0, 2, :]
    for t in range(K_NN):
        r0, r1, r2, j0, j1, j2 = _insert(pv_ref[1, t, :], pi_ref[1, t, :],
                                         r0, r1, r2, j0, j1, j2)
    vals_ref[...] = jnp.stack([r0, r1, r2], axis=1)
    idx_ref[...] = jnp.stack([j0, j1, j2], axis=1)


def kernel(queries, memory_bank):
    q_n, dim = queries.shape
    n_total, _ = memory_bank.shape
    block_n = min(BLOCK_N, n_total)
    nblocks = n_total // block_n
    cores = N_CORES if nblocks % N_CORES == 0 else 1
    inner = nblocks // cores

    pv, pi = pl.pallas_call(
        functools.partial(_knn_step, block_n=block_n, n_total=n_total,
                          inner=inner),
        grid=(cores, inner),
        in_specs=[
            pl.BlockSpec((q_n, dim), lambda i, j: (0, 0)),
            pl.BlockSpec((block_n, dim), lambda i, j: (i * inner + j, 0)),
        ],
        out_specs=[
            pl.BlockSpec((1, 8, q_n), lambda i, j: (i, 0, 0)),
            pl.BlockSpec((1, 8, q_n), lambda i, j: (i, 0, 0)),
        ],
        out_shape=[
            jax.ShapeDtypeStruct((cores, 8, q_n), jnp.float32),
            jax.ShapeDtypeStruct((cores, 8, q_n), jnp.int32),
        ],
        scratch_shapes=[
            pltpu.VMEM((8, q_n), jnp.float32),
            pltpu.VMEM((8, q_n), jnp.int32),
        ],
        compiler_params=pltpu.CompilerParams(
            dimension_semantics=("parallel", "arbitrary"),
        ),
    )(queries, memory_bank)

    if cores == 1:
        vals = jnp.transpose(pv[0, :K_NN, :])
        idx = jnp.transpose(pi[0, :K_NN, :])
        return vals, idx

    vals, idx = pl.pallas_call(
        _merge_step,
        out_shape=[
            jax.ShapeDtypeStruct((q_n, K_NN), jnp.float32),
            jax.ShapeDtypeStruct((q_n, K_NN), jnp.int32),
        ],
    )(pv, pi)
    return vals, idx


# speedup vs baseline: 1.8154x; 1.8154x over previous
import functools

import jax
import jax.numpy as jnp
from jax.experimental import pallas as pl
from jax.experimental.pallas import tpu as pltpu

K_NN = 3
BLOCK_N = 1024
LANES = 128


def _insert(v, ix, s0, s1, s2, i0, i1, i2):
    b0 = v < s0
    b1 = v < s1
    b2 = v < s2
    return (
        jnp.where(b0, v, s0),
        jnp.where(b0, s0, jnp.where(b1, v, s1)),
        jnp.where(b1, s1, jnp.where(b2, v, s2)),
        jnp.where(b0, ix, i0),
        jnp.where(b0, i0, jnp.where(b1, ix, i1)),
        jnp.where(b1, i1, jnp.where(b2, ix, i2)),
    )


def _knn_step(q_ref, m_ref, vals_ref, idx_ref,
              qs_ref, s0_ref, s1_ref, s2_ref, i0_ref, i1_ref, i2_ref, *,
              block_n, n_total):
    j = pl.program_id(0)
    nsteps = pl.num_programs(0)
    q_n = q_ref.shape[0]

    @pl.when(j == 0)
    def _init():
        q = q_ref[...]
        qs_ref[0, :] = jnp.sum(q * q, axis=1)
        inf = jnp.full((q_n, LANES), jnp.inf, jnp.float32)
        zero = jnp.zeros((q_n, LANES), jnp.int32)
        s0_ref[...] = inf
        s1_ref[...] = inf
        s2_ref[...] = inf
        i0_ref[...] = zero
        i1_ref[...] = zero
        i2_ref[...] = zero

    q = q_ref[...]
    m = m_ref[...]
    qsq = qs_ref[0, :]
    msq = jnp.sum(m * m, axis=1)
    ab = jax.lax.dot_general(q, m, (((1,), (1,)), ((), ())),
                             preferred_element_type=jnp.float32)
    d2 = (qsq[:, None] + msq[None, :]) - 2.0 * ab
    dist = jnp.sqrt(jnp.maximum(d2, 1e-12))

    lane = jax.lax.broadcasted_iota(jnp.int32, (q_n, LANES), 1)
    base = j * block_n

    s0 = s0_ref[...]
    s1 = s1_ref[...]
    s2 = s2_ref[...]
    i0 = i0_ref[...]
    i1 = i1_ref[...]
    i2 = i2_ref[...]

    for g in range(block_n // LANES):
        v = dist[:, g * LANES:(g + 1) * LANES]
        ix = lane + (base + g * LANES)
        s0, s1, s2, i0, i1, i2 = _insert(v, ix, s0, s1, s2, i0, i1, i2)

    s0_ref[...] = s0
    s1_ref[...] = s1
    s2_ref[...] = s2
    i0_ref[...] = i0
    i1_ref[...] = i1
    i2_ref[...] = i2

    @pl.when(j == nsteps - 1)
    def _finish():
        a0, a1, a2 = s0, s1, s2
        b0, b1, b2 = i0, i1, i2
        out_v = []
        out_i = []
        for _ in range(K_NN):
            vk = jnp.min(a0, axis=1)
            hit = a0 == vk[:, None]
            jk = jnp.min(jnp.where(hit, b0, n_total), axis=1)
            sel = hit & (b0 == jk[:, None])
            a0 = jnp.where(sel, a1, a0)
            b0 = jnp.where(sel, b1, b0)
            a1 = jnp.where(sel, a2, a1)
            b1 = jnp.where(sel, b2, b1)
            a2 = jnp.where(sel, jnp.inf, a2)
            out_v.append(vk)
            out_i.append(jk)
        vals_ref[...] = jnp.stack(out_v, axis=1)
        idx_ref[...] = jnp.stack(out_i, axis=1)


def kernel(queries, memory_bank):
    q_n, dim = queries.shape
    n_total, _ = memory_bank.shape
    block_n = min(BLOCK_N, n_total)
    grid = n_total // block_n

    vals, idx = pl.pallas_call(
        functools.partial(_knn_step, block_n=block_n, n_total=n_total),
        grid=(grid,),
        in_specs=[
            pl.BlockSpec((q_n, dim), lambda j: (0, 0)),
            pl.BlockSpec((block_n, dim), lambda j: (j, 0)),
        ],
        out_specs=[
            pl.BlockSpec((q_n, K_NN), lambda j: (0, 0)),
            pl.BlockSpec((q_n, K_NN), lambda j: (0, 0)),
        ],
        out_shape=[
            jax.ShapeDtypeStruct((q_n, K_NN), jnp.float32),
            jax.ShapeDtypeStruct((q_n, K_NN), jnp.int32),
        ],
        scratch_shapes=[
            pltpu.VMEM((8, q_n), jnp.float32),
            pltpu.VMEM((q_n, LANES), jnp.float32),
            pltpu.VMEM((q_n, LANES), jnp.float32),
            pltpu.VMEM((q_n, LANES), jnp.float32),
            pltpu.VMEM((q_n, LANES), jnp.int32),
            pltpu.VMEM((q_n, LANES), jnp.int32),
            pltpu.VMEM((q_n, LANES), jnp.int32),
        ],
    )(queries, memory_bank)
    return vals, idx


# speedup vs baseline: 1.8283x; 1.0071x over previous
import functools

import jax
import jax.numpy as jnp
from jax.experimental import pallas as pl
from jax.experimental.pallas import tpu as pltpu

K_NN = 3
BLOCK_N = 1024
LANES = 128
CHUNK = 56


def _insert(v, ix, s0, s1, s2, i0, i1, i2):
    b0 = v < s0
    b1 = v < s1
    b2 = v < s2
    return (
        jnp.where(b0, v, s0),
        jnp.where(b0, s0, jnp.where(b1, v, s1)),
        jnp.where(b1, s1, jnp.where(b2, v, s2)),
        jnp.where(b0, ix, i0),
        jnp.where(b0, i0, jnp.where(b1, ix, i1)),
        jnp.where(b1, i1, jnp.where(b2, ix, i2)),
    )


def _knn_step(q_ref, m_ref, vals_ref, idx_ref,
              qs_ref, s0_ref, s1_ref, s2_ref, i0_ref, i1_ref, i2_ref, *,
              block_n, n_total):
    j = pl.program_id(0)
    nsteps = pl.num_programs(0)
    q_n = q_ref.shape[0]

    @pl.when(j == 0)
    def _init():
        q = q_ref[...]
        qs_ref[0, :] = jnp.sum(q * q, axis=1)
        inf = jnp.full((q_n, LANES), jnp.inf, jnp.float32)
        zero = jnp.zeros((q_n, LANES), jnp.int32)
        s0_ref[...] = inf
        s1_ref[...] = inf
        s2_ref[...] = inf
        i0_ref[...] = zero
        i1_ref[...] = zero
        i2_ref[...] = zero

    q = q_ref[...]
    m = m_ref[...]
    qsq = qs_ref[0, :]
    dim = m.shape[1]
    mm = m * m
    macc = mm[:, :LANES]
    for g in range(1, dim // LANES):
        macc = macc + mm[:, g * LANES:(g + 1) * LANES]
    msq = jnp.sum(macc, axis=1)
    ab = jax.lax.dot_general(q, m, (((1,), (1,)), ((), ())),
                             preferred_element_type=jnp.float32)
    d2 = (qsq[:, None] + msq[None, :]) - 2.0 * ab
    dist = jnp.sqrt(jnp.maximum(d2, 1e-12))

    chunk = CHUNK if q_n % CHUNK == 0 else q_n
    lane = jax.lax.broadcasted_iota(jnp.int32, (chunk, LANES), 1)
    base = j * block_n

    for c in range(q_n // chunk):
        rows = slice(c * chunk, (c + 1) * chunk)
        s0 = s0_ref[rows, :]
        s1 = s1_ref[rows, :]
        s2 = s2_ref[rows, :]
        i0 = i0_ref[rows, :]
        i1 = i1_ref[rows, :]
        i2 = i2_ref[rows, :]
        for g in range(block_n // LANES):
            v = dist[rows, g * LANES:(g + 1) * LANES]
            ix = lane + (base + g * LANES)
            s0, s1, s2, i0, i1, i2 = _insert(v, ix, s0, s1, s2, i0, i1, i2)
        s0_ref[rows, :] = s0
        s1_ref[rows, :] = s1
        s2_ref[rows, :] = s2
        i0_ref[rows, :] = i0
        i1_ref[rows, :] = i1
        i2_ref[rows, :] = i2

    @pl.when(j == nsteps - 1)
    def _finish():
        a0, a1, a2 = s0_ref[...], s1_ref[...], s2_ref[...]
        b0, b1, b2 = i0_ref[...], i1_ref[...], i2_ref[...]
        out_v = []
        out_i = []
        for _ in range(K_NN):
            vk = jnp.min(a0, axis=1)
            hit = a0 == vk[:, None]
            jk = jnp.min(jnp.where(hit, b0, n_total), axis=1)
            sel = hit & (b0 == jk[:, None])
            a0 = jnp.where(sel, a1, a0)
            b0 = jnp.where(sel, b1, b0)
            a1 = jnp.where(sel, a2, a1)
            b1 = jnp.where(sel, b2, b1)
            a2 = jnp.where(sel, jnp.inf, a2)
            out_v.append(vk)
            out_i.append(jk)
        vals_ref[...] = jnp.stack(out_v, axis=1)
        idx_ref[...] = jnp.stack(out_i, axis=1)


def kernel(queries, memory_bank):
    q_n, dim = queries.shape
    n_total, _ = memory_bank.shape
    block_n = min(BLOCK_N, n_total)
    grid = n_total // block_n

    vals, idx = pl.pallas_call(
        functools.partial(_knn_step, block_n=block_n, n_total=n_total),
        grid=(grid,),
        in_specs=[
            pl.BlockSpec((q_n, dim), lambda j: (0, 0)),
            pl.BlockSpec((block_n, dim), lambda j: (j, 0)),
        ],
        out_specs=[
            pl.BlockSpec((q_n, K_NN), lambda j: (0, 0)),
            pl.BlockSpec((q_n, K_NN), lambda j: (0, 0)),
        ],
        out_shape=[
            jax.ShapeDtypeStruct((q_n, K_NN), jnp.float32),
            jax.ShapeDtypeStruct((q_n, K_NN), jnp.int32),
        ],
        scratch_shapes=[
            pltpu.VMEM((8, q_n), jnp.float32),
            pltpu.VMEM((q_n, LANES), jnp.float32),
            pltpu.VMEM((q_n, LANES), jnp.float32),
            pltpu.VMEM((q_n, LANES), jnp.float32),
            pltpu.VMEM((q_n, LANES), jnp.int32),
            pltpu.VMEM((q_n, LANES), jnp.int32),
            pltpu.VMEM((q_n, LANES), jnp.int32),
        ],
    )(queries, memory_bank)
    return vals, idx


# speedup vs baseline: 2.2372x; 1.2237x over previous
import functools

import jax
import jax.numpy as jnp
from jax.experimental import pallas as pl
from jax.experimental.pallas import tpu as pltpu

K_NN = 3
BLOCK_N = 1024
LANES = 128
CHUNK = 56


def _insert(v, ix, s0, s1, s2, i0, i1, i2):
    b0 = v < s0
    b1 = v < s1
    b2 = v < s2
    m0 = jnp.maximum(s0, v)
    m1 = jnp.maximum(s1, m0)
    return (
        jnp.minimum(s0, v),
        jnp.minimum(s1, m0),
        jnp.minimum(s2, m1),
        jnp.where(b0, ix, i0),
        jnp.where(b0, i0, jnp.where(b1, ix, i1)),
        jnp.where(b1, i1, jnp.where(b2, ix, i2)),
    )


def _knn_step(q_ref, m_ref, vals_ref, idx_ref,
              qs_ref, s0_ref, s1_ref, s2_ref, i0_ref, i1_ref, i2_ref, *,
              block_n, n_total):
    j = pl.program_id(0)
    nsteps = pl.num_programs(0)
    q_n = q_ref.shape[0]

    @pl.when(j == 0)
    def _init():
        q = q_ref[...]
        qs_ref[0, :] = jnp.sum(q * q, axis=1)
        inf = jnp.full((q_n, LANES), jnp.inf, jnp.float32)
        zero = jnp.zeros((q_n, LANES), jnp.int32)
        s0_ref[...] = inf
        s1_ref[...] = inf
        s2_ref[...] = inf
        i0_ref[...] = zero
        i1_ref[...] = zero
        i2_ref[...] = zero

    q = q_ref[...]
    m = m_ref[...]
    qsq = qs_ref[0, :]
    dim = m.shape[1]
    mm = m * m
    macc = mm[:, :LANES]
    for g in range(1, dim // LANES):
        macc = macc + mm[:, g * LANES:(g + 1) * LANES]
    msq = jnp.sum(macc, axis=1)
    ab = jax.lax.dot_general(q, m, (((1,), (1,)), ((), ())),
                             preferred_element_type=jnp.float32)
    d2 = (qsq[:, None] + msq[None, :]) - 2.0 * ab

    chunk = CHUNK if q_n % CHUNK == 0 else q_n
    lane = jax.lax.broadcasted_iota(jnp.int32, (chunk, LANES), 1)
    base = j * block_n

    for c in range(q_n // chunk):
        rows = slice(c * chunk, (c + 1) * chunk)
        s0 = s0_ref[rows, :]
        s1 = s1_ref[rows, :]
        s2 = s2_ref[rows, :]
        i0 = i0_ref[rows, :]
        i1 = i1_ref[rows, :]
        i2 = i2_ref[rows, :]
        for g in range(block_n // LANES):
            v = d2[rows, g * LANES:(g + 1) * LANES]
            ix = lane + (base + g * LANES)
            s0, s1, s2, i0, i1, i2 = _insert(v, ix, s0, s1, s2, i0, i1, i2)
        s0_ref[rows, :] = s0
        s1_ref[rows, :] = s1
        s2_ref[rows, :] = s2
        i0_ref[rows, :] = i0
        i1_ref[rows, :] = i1
        i2_ref[rows, :] = i2

    @pl.when(j == nsteps - 1)
    def _finish():
        a0 = jnp.sqrt(jnp.maximum(s0_ref[...], 1e-12))
        a1 = jnp.sqrt(jnp.maximum(s1_ref[...], 1e-12))
        a2 = jnp.sqrt(jnp.maximum(s2_ref[...], 1e-12))
        b0, b1, b2 = i0_ref[...], i1_ref[...], i2_ref[...]
        c = (a0 == a1) & (b1 < b0)
        b0, b1 = jnp.where(c, b1, b0), jnp.where(c, b0, b1)
        c = (a1 == a2) & (b2 < b1)
        b1, b2 = jnp.where(c, b2, b1), jnp.where(c, b1, b2)
        c = (a0 == a1) & (b1 < b0)
        b0, b1 = jnp.where(c, b1, b0), jnp.where(c, b0, b1)
        out_v = []
        out_i = []
        for _ in range(K_NN):
            vk = jnp.min(a0, axis=1)
            hit = a0 == vk[:, None]
            jk = jnp.min(jnp.where(hit, b0, n_total), axis=1)
            sel = hit & (b0 == jk[:, None])
            a0 = jnp.where(sel, a1, a0)
            b0 = jnp.where(sel, b1, b0)
            a1 = jnp.where(sel, a2, a1)
            b1 = jnp.where(sel, b2, b1)
            a2 = jnp.where(sel, jnp.inf, a2)
            out_v.append(vk)
            out_i.append(jk)
        vals_ref[...] = jnp.stack(out_v, axis=1)
        idx_ref[...] = jnp.stack(out_i, axis=1)


def kernel(queries, memory_bank):
    q_n, dim = queries.shape
    n_total, _ = memory_bank.shape
    block_n = min(BLOCK_N, n_total)
    grid = n_total // block_n

    vals, idx = pl.pallas_call(
        functools.partial(_knn_step, block_n=block_n, n_total=n_total),
        grid=(grid,),
        in_specs=[
            pl.BlockSpec((q_n, dim), lambda j: (0, 0)),
            pl.BlockSpec((block_n, dim), lambda j: (j, 0)),
        ],
        out_specs=[
            pl.BlockSpec((q_n, K_NN), lambda j: (0, 0)),
            pl.BlockSpec((q_n, K_NN), lambda j: (0, 0)),
        ],
        out_shape=[
            jax.ShapeDtypeStruct((q_n, K_NN), jnp.float32),
            jax.ShapeDtypeStruct((q_n, K_NN), jnp.int32),
        ],
        scratch_shapes=[
            pltpu.VMEM((8, q_n), jnp.float32),
            pltpu.VMEM((q_n, LANES), jnp.float32),
            pltpu.VMEM((q_n, LANES), jnp.float32),
            pltpu.VMEM((q_n, LANES), jnp.float32),
            pltpu.VMEM((q_n, LANES), jnp.int32),
            pltpu.VMEM((q_n, LANES), jnp.int32),
            pltpu.VMEM((q_n, LANES), jnp.int32),
        ],
    )(queries, memory_bank)
    return vals, idx


# speedup vs baseline: 2.2666x; 1.0131x over previous
import functools

import jax
import jax.numpy as jnp
from jax.experimental import pallas as pl
from jax.experimental.pallas import tpu as pltpu

K_NN = 3
BLOCK_N = 1024
LANES = 128
CHUNK = 56


def _insert(v, ix, s0, s1, s2, i0, i1, i2):
    b0 = v < s0
    b1 = v < s1
    b2 = v < s2
    m0 = jnp.maximum(s0, v)
    m1 = jnp.maximum(s1, m0)
    return (
        jnp.minimum(s0, v),
        jnp.minimum(s1, m0),
        jnp.minimum(s2, m1),
        jnp.where(b0, ix, i0),
        jnp.where(b0, i0, jnp.where(b1, ix, i1)),
        jnp.where(b1, i1, jnp.where(b2, ix, i2)),
    )


def _knn_step(q_ref, m_ref, vals_ref, idx_ref,
              qs_ref, s0_ref, s1_ref, s2_ref, i0_ref, i1_ref, i2_ref, *,
              block_n, n_total):
    j = pl.program_id(0)
    nsteps = pl.num_programs(0)
    q_n = q_ref.shape[0]

    @pl.when(j == 0)
    def _init():
        q = q_ref[...]
        qsq = jnp.sum(q * q, axis=1)
        qs_ref[...] = jnp.broadcast_to(qsq[:, None], (q_n, LANES))
        inf = jnp.full((q_n, LANES), jnp.inf, jnp.float32)
        zero = jnp.zeros((q_n, LANES), jnp.int32)
        s0_ref[...] = inf
        s1_ref[...] = inf
        s2_ref[...] = inf
        i0_ref[...] = zero
        i1_ref[...] = zero
        i2_ref[...] = zero

    q = q_ref[...]
    m = m_ref[...]
    dim = m.shape[1]
    mm = m * m
    macc = mm[:, :LANES]
    for g in range(1, dim // LANES):
        macc = macc + mm[:, g * LANES:(g + 1) * LANES]
    msq = jnp.sum(macc, axis=1)
    ab = jax.lax.dot_general(q, m, (((1,), (1,)), ((), ())),
                             preferred_element_type=jnp.float32)

    chunk = CHUNK if q_n % CHUNK == 0 else q_n
    lane = jax.lax.broadcasted_iota(jnp.int32, (chunk, LANES), 1)
    base = j * block_n

    for c in range(q_n // chunk):
        rows = slice(c * chunk, (c + 1) * chunk)
        qsqb = qs_ref[rows, :]
        s0 = s0_ref[rows, :]
        s1 = s1_ref[rows, :]
        s2 = s2_ref[rows, :]
        i0 = i0_ref[rows, :]
        i1 = i1_ref[rows, :]
        i2 = i2_ref[rows, :]
        for g in range(block_n // LANES):
            cols = slice(g * LANES, (g + 1) * LANES)
            v = (qsqb + msq[None, cols]) - 2.0 * ab[rows, cols]
            ix = lane + (base + g * LANES)
            s0, s1, s2, i0, i1, i2 = _insert(v, ix, s0, s1, s2, i0, i1, i2)
        s0_ref[rows, :] = s0
        s1_ref[rows, :] = s1
        s2_ref[rows, :] = s2
        i0_ref[rows, :] = i0
        i1_ref[rows, :] = i1
        i2_ref[rows, :] = i2

    @pl.when(j == nsteps - 1)
    def _finish():
        a0 = jnp.sqrt(jnp.maximum(s0_ref[...], 1e-12))
        a1 = jnp.sqrt(jnp.maximum(s1_ref[...], 1e-12))
        a2 = jnp.sqrt(jnp.maximum(s2_ref[...], 1e-12))
        b0, b1, b2 = i0_ref[...], i1_ref[...], i2_ref[...]
        c = (a0 == a1) & (b1 < b0)
        b0, b1 = jnp.where(c, b1, b0), jnp.where(c, b0, b1)
        c = (a1 == a2) & (b2 < b1)
        b1, b2 = jnp.where(c, b2, b1), jnp.where(c, b1, b2)
        c = (a0 == a1) & (b1 < b0)
        b0, b1 = jnp.where(c, b1, b0), jnp.where(c, b0, b1)
        out_v = []
        out_i = []
        for _ in range(K_NN):
            vk = jnp.min(a0, axis=1)
            hit = a0 == vk[:, None]
            jk = jnp.min(jnp.where(hit, b0, n_total), axis=1)
            sel = hit & (b0 == jk[:, None])
            a0 = jnp.where(sel, a1, a0)
            b0 = jnp.where(sel, b1, b0)
            a1 = jnp.where(sel, a2, a1)
            b1 = jnp.where(sel, b2, b1)
            a2 = jnp.where(sel, jnp.inf, a2)
            out_v.append(vk)
            out_i.append(jk)
        vals_ref[...] = jnp.stack(out_v, axis=1)
        idx_ref[...] = jnp.stack(out_i, axis=1)


def kernel(queries, memory_bank):
    q_n, dim = queries.shape
    n_total, _ = memory_bank.shape
    block_n = min(BLOCK_N, n_total)
    grid = n_total // block_n

    vals, idx = pl.pallas_call(
        functools.partial(_knn_step, block_n=block_n, n_total=n_total),
        grid=(grid,),
        in_specs=[
            pl.BlockSpec((q_n, dim), lambda j: (0, 0)),
            pl.BlockSpec((block_n, dim), lambda j: (j, 0)),
        ],
        out_specs=[
            pl.BlockSpec((q_n, K_NN), lambda j: (0, 0)),
            pl.BlockSpec((q_n, K_NN), lambda j: (0, 0)),
        ],
        out_shape=[
            jax.ShapeDtypeStruct((q_n, K_NN), jnp.float32),
            jax.ShapeDtypeStruct((q_n, K_NN), jnp.int32),
        ],
        scratch_shapes=[
            pltpu.VMEM((q_n, LANES), jnp.float32),
            pltpu.VMEM((q_n, LANES), jnp.float32),
            pltpu.VMEM((q_n, LANES), jnp.float32),
            pltpu.VMEM((q_n, LANES), jnp.float32),
            pltpu.VMEM((q_n, LANES), jnp.int32),
            pltpu.VMEM((q_n, LANES), jnp.int32),
            pltpu.VMEM((q_n, LANES), jnp.int32),
        ],
    )(queries, memory_bank)
    return vals, idx


# speedup vs baseline: 2.3742x; 1.0475x over previous
import functools

import jax
import jax.numpy as jnp
from jax.experimental import pallas as pl
from jax.experimental.pallas import tpu as pltpu

K_NN = 3
BLOCK_N = 2048
LANES = 128
CHUNK = 56


def _insert(v, ix, s0, s1, s2, i0, i1, i2):
    b0 = v < s0
    b1 = v < s1
    b2 = v < s2
    m0 = jnp.maximum(s0, v)
    m1 = jnp.maximum(s1, m0)
    return (
        jnp.minimum(s0, v),
        jnp.minimum(s1, m0),
        jnp.minimum(s2, m1),
        jnp.where(b0, ix, i0),
        jnp.where(b0, i0, jnp.where(b1, ix, i1)),
        jnp.where(b1, i1, jnp.where(b2, ix, i2)),
    )


def _knn_step(q_ref, m_ref, vals_ref, idx_ref,
              qs_ref, q2_ref, s0_ref, s1_ref, s2_ref,
              i0_ref, i1_ref, i2_ref, *,
              block_n, n_total):
    j = pl.program_id(0)
    nsteps = pl.num_programs(0)
    q_n = q_ref.shape[0]

    @pl.when(j == 0)
    def _init():
        q = q_ref[...]
        qsq = jnp.sum(q * q, axis=1)
        qs_ref[...] = jnp.broadcast_to(qsq[:, None], (q_n, LANES))
        q2_ref[...] = q + q
        inf = jnp.full((q_n, LANES), jnp.inf, jnp.float32)
        zero = jnp.zeros((q_n, LANES), jnp.int32)
        s0_ref[...] = inf
        s1_ref[...] = inf
        s2_ref[...] = inf
        i0_ref[...] = zero
        i1_ref[...] = zero
        i2_ref[...] = zero

    m = m_ref[...]
    dim = m.shape[1]
    mm = m * m
    macc = mm[:, :LANES]
    for g in range(1, dim // LANES):
        macc = macc + mm[:, g * LANES:(g + 1) * LANES]
    msq = jnp.sum(macc, axis=1)
    ab2 = jax.lax.dot_general(q2_ref[...], m, (((1,), (1,)), ((), ())),
                              preferred_element_type=jnp.float32)

    chunk = CHUNK if q_n % CHUNK == 0 else q_n
    lane = jax.lax.broadcasted_iota(jnp.int32, (chunk, LANES), 1)
    base = j * block_n

    for c in range(q_n // chunk):
        rows = slice(c * chunk, (c + 1) * chunk)
        qsqb = qs_ref[rows, :]
        s0 = s0_ref[rows, :]
        s1 = s1_ref[rows, :]
        s2 = s2_ref[rows, :]
        i0 = i0_ref[rows, :]
        i1 = i1_ref[rows, :]
        i2 = i2_ref[rows, :]
        for g in range(block_n // LANES):
            cols = slice(g * LANES, (g + 1) * LANES)
            v = (qsqb + msq[None, cols]) - ab2[rows, cols]
            ix = lane + (base + g * LANES)
            s0, s1, s2, i0, i1, i2 = _insert(v, ix, s0, s1, s2, i0, i1, i2)
        s0_ref[rows, :] = s0
        s1_ref[rows, :] = s1
        s2_ref[rows, :] = s2
        i0_ref[rows, :] = i0
        i1_ref[rows, :] = i1
        i2_ref[rows, :] = i2

    @pl.when(j == nsteps - 1)
    def _finish():
        a0 = jnp.sqrt(jnp.maximum(s0_ref[...], 1e-12))
        a1 = jnp.sqrt(jnp.maximum(s1_ref[...], 1e-12))
        a2 = jnp.sqrt(jnp.maximum(s2_ref[...], 1e-12))
        b0, b1, b2 = i0_ref[...], i1_ref[...], i2_ref[...]
        c = (a0 == a1) & (b1 < b0)
        b0, b1 = jnp.where(c, b1, b0), jnp.where(c, b0, b1)
        c = (a1 == a2) & (b2 < b1)
        b1, b2 = jnp.where(c, b2, b1), jnp.where(c, b1, b2)
        c = (a0 == a1) & (b1 < b0)
        b0, b1 = jnp.where(c, b1, b0), jnp.where(c, b0, b1)
        out_v = []
        out_i = []
        for _ in range(K_NN):
            vk = jnp.min(a0, axis=1)
            hit = a0 == vk[:, None]
            jk = jnp.min(jnp.where(hit, b0, n_total), axis=1)
            sel = hit & (b0 == jk[:, None])
            a0 = jnp.where(sel, a1, a0)
            b0 = jnp.where(sel, b1, b0)
            a1 = jnp.where(sel, a2, a1)
            b1 = jnp.where(sel, b2, b1)
            a2 = jnp.where(sel, jnp.inf, a2)
            out_v.append(vk)
            out_i.append(jk)
        vals_ref[...] = jnp.stack(out_v, axis=1)
        idx_ref[...] = jnp.stack(out_i, axis=1)


def kernel(queries, memory_bank):
    q_n, dim = queries.shape
    n_total, _ = memory_bank.shape
    block_n = min(BLOCK_N, n_total)
    grid = n_total // block_n

    vals, idx = pl.pallas_call(
        functools.partial(_knn_step, block_n=block_n, n_total=n_total),
        grid=(grid,),
        in_specs=[
            pl.BlockSpec((q_n, dim), lambda j: (0, 0)),
            pl.BlockSpec((block_n, dim), lambda j: (j, 0)),
        ],
        out_specs=[
            pl.BlockSpec((q_n, K_NN), lambda j: (0, 0)),
            pl.BlockSpec((q_n, K_NN), lambda j: (0, 0)),
        ],
        out_shape=[
            jax.ShapeDtypeStruct((q_n, K_NN), jnp.float32),
            jax.ShapeDtypeStruct((q_n, K_NN), jnp.int32),
        ],
        scratch_shapes=[
            pltpu.VMEM((q_n, LANES), jnp.float32),
            pltpu.VMEM((q_n, dim), jnp.float32),
            pltpu.VMEM((q_n, LANES), jnp.float32),
            pltpu.VMEM((q_n, LANES), jnp.float32),
            pltpu.VMEM((q_n, LANES), jnp.float32),
            pltpu.VMEM((q_n, LANES), jnp.int32),
            pltpu.VMEM((q_n, LANES), jnp.int32),
            pltpu.VMEM((q_n, LANES), jnp.int32),
        ],
    )(queries, memory_bank)
    return vals, idx
